# baseline (speedup 1.0000x reference)
import functools

import jax
import jax.numpy as jnp
from jax import lax
from jax.experimental import pallas as pl
from jax.experimental.pallas import tpu as pltpu
from jax.experimental.pallas import tpu_sc as plsc

BATCH = 16384
EMBED_DIM = 64

_NC = 2
_NS = 16
_NW = _NC * _NS
_BPW = BATCH // _NW
_CHUNK = 128
_NCHUNK = _BPW // _CHUNK


@functools.partial(
    pl.kernel,
    mesh=plsc.VectorSubcoreMesh(core_axis_name="c", subcore_axis_name="s"),
    out_type=(
        jax.ShapeDtypeStruct((BATCH, EMBED_DIM), jnp.float32),
        jax.ShapeDtypeStruct((BATCH, EMBED_DIM), jnp.float32),
    ),
    scratch_types=[
        pltpu.VMEM((_NCHUNK, _CHUNK), jnp.int32),
        pltpu.VMEM((_NCHUNK, _CHUNK), jnp.int32),
        pltpu.VMEM((_BPW, EMBED_DIM), jnp.float32),
        pltpu.VMEM((_BPW, EMBED_DIM), jnp.float32),
        pltpu.SemaphoreType.DMA,
        pltpu.SemaphoreType.DMA,
        pltpu.SemaphoreType.DMA,
    ],
    compiler_params=pltpu.CompilerParams(use_tc_tiling_on_sc=False),
)
def _gather2(uids, iids, utab, itab, uout, iout,
             idx_u, idx_i, rows_u, rows_i, sem_u, sem_i, sem_w):
    wid = lax.axis_index("s") * _NC + lax.axis_index("c")
    base = wid * _BPW
    pltpu.sync_copy(uids.at[wid], idx_u)
    pltpu.sync_copy(iids.at[wid], idx_i)
    ucps = [
        pltpu.async_copy(utab.at[idx_u.at[j]],
                         rows_u.at[pl.ds(j * _CHUNK, _CHUNK)], sem_u)
        for j in range(_NCHUNK)
    ]
    icps = [
        pltpu.async_copy(itab.at[idx_i.at[j]],
                         rows_i.at[pl.ds(j * _CHUNK, _CHUNK)], sem_i)
        for j in range(_NCHUNK)
    ]
    for c in ucps:
        c.wait()
    wu = pltpu.async_copy(rows_u, uout.at[pl.ds(base, _BPW)], sem_w)
    for c in icps:
        c.wait()
    wi = pltpu.async_copy(rows_i, iout.at[pl.ds(base, _BPW)], sem_w)
    wu.wait()
    wi.wait()


def kernel(user_ids, item_ids, user_table, item_table):
    uids = user_ids.reshape(_NW, _NCHUNK, _CHUNK)
    iids = item_ids.reshape(_NW, _NCHUNK, _CHUNK)
    return _gather2(uids, iids, user_table, item_table)


# speedup vs baseline: 2.4375x; 2.4375x over previous
import functools

import jax
import jax.numpy as jnp
from jax import lax
from jax.experimental import pallas as pl
from jax.experimental.pallas import tpu as pltpu
from jax.experimental.pallas import tpu_sc as plsc

BATCH = 16384
EMBED_DIM = 64
LANES = 16

_NC = 2
_NS = 16
_NW = _NC * _NS
_BPW = BATCH // _NW
_NRING = 4
_NCHUNK = _BPW // LANES


def _gather_one(tabT, vidx, stage, cols, sems, dummy_src):
    def enqueue(i, q):
        tc = pl.multiple_of((i >> 7) * 128, 128)
        pltpu.async_copy(tabT.at[:, pl.ds(tc, 128)], stage.at[q], sems[q])

    def extract(i, c, q):
        col = i & 127
        cvec = jnp.full((LANES,), col, jnp.int32)
        jvec = jnp.full((LANES,), c, jnp.int32)
        for b in range(EMBED_DIM // LANES):
            dvec = lax.iota(jnp.int32, LANES) + b * LANES
            v = plsc.load_gather(stage.at[q], [dvec, cvec])
            plsc.store_scatter(cols, [dvec, jvec], v)

    v0 = vidx[pl.ds(0, LANES)]
    for q in range(_NRING):
        enqueue(v0[q], q)

    def body(k, _):
        off = k * LANES
        v = vidx[pl.ds(off, LANES)]
        offn = jnp.minimum(off + LANES, _BPW - LANES)
        vn = vidx[pl.ds(offn, LANES)]
        for j in range(LANES):
            q = j % _NRING
            pltpu.make_async_copy(dummy_src, stage.at[q], sems[q]).wait()
            extract(v[j], off + j, q)
            nxt = v[j + _NRING] if j < LANES - _NRING else vn[j - (LANES - _NRING)]

            @pl.when(off + j + _NRING < _BPW)
            def _():
                enqueue(nxt, q)

        return ()

    lax.fori_loop(0, _NCHUNK, body, (), unroll=False)


@functools.partial(
    pl.kernel,
    mesh=plsc.VectorSubcoreMesh(core_axis_name="c", subcore_axis_name="s"),
    out_type=(
        jax.ShapeDtypeStruct((EMBED_DIM, BATCH), jnp.float32),
        jax.ShapeDtypeStruct((EMBED_DIM, BATCH), jnp.float32),
    ),
    scratch_types=[
        pltpu.VMEM((_BPW,), jnp.int32),
        pltpu.VMEM((_BPW,), jnp.int32),
        pltpu.VMEM((_NRING, EMBED_DIM, 128), jnp.float32),
        pltpu.VMEM((EMBED_DIM, _BPW), jnp.float32),
        pltpu.VMEM((EMBED_DIM, _BPW), jnp.float32),
        [pltpu.SemaphoreType.DMA] * _NRING,
        pltpu.SemaphoreType.DMA,
    ],
    compiler_params=pltpu.CompilerParams(needs_layout_passes=False),
)
def _gather2(uids, iids, utabT, itabT, uoutT, ioutT,
             vidx_u, vidx_i, stage, cols_u, cols_i, sems, sem_w):
    wid = lax.axis_index("s") * _NC + lax.axis_index("c")
    base = wid * _BPW
    pltpu.sync_copy(uids.at[pl.ds(base, _BPW)], vidx_u)
    pltpu.sync_copy(iids.at[pl.ds(base, _BPW)], vidx_i)

    dummy_src = utabT.at[:, pl.ds(0, 128)]
    _gather_one(utabT, vidx_u, stage, cols_u, sems, dummy_src)
    wu = pltpu.async_copy(cols_u, uoutT.at[:, pl.ds(base, _BPW)], sem_w)
    _gather_one(itabT, vidx_i, stage, cols_i, sems, dummy_src)
    wi = pltpu.async_copy(cols_i, ioutT.at[:, pl.ds(base, _BPW)], sem_w)
    wu.wait()
    wi.wait()


def kernel(user_ids, item_ids, user_table, item_table):
    uT, iT = _gather2(user_ids, item_ids, user_table.T, item_table.T)
    return (uT.T, iT.T)


# speedup vs baseline: 3.4265x; 1.4058x over previous
import functools

import jax
import jax.numpy as jnp
from jax import lax
from jax.experimental import pallas as pl
from jax.experimental.pallas import tpu as pltpu
from jax.experimental.pallas import tpu_sc as plsc

BATCH = 16384
EMBED_DIM = 64
LANES = 16

_NC = 2
_NS = 16
_NW = _NC * _NS
_BPW = BATCH // _NW
_NTC = 7813
_TPW = 245
_NGRP = (_TPW + 3) // 4
_NRING = 4
_WRING = 8
_NVEC = BATCH // LANES


def _extract_scalar(ref, pos):
    return ref[pl.ds(pos, LANES)][0]


def _gather_one(wid, allids, tabT, exch, clist_id, clist_pos,
                sort_id, sort_pos, begin_v, end_v, stage, tmp,
                sems, sem_w):
    iota = lax.iota(jnp.int32, LANES)
    ones = jnp.ones((LANES,), jnp.int32)

    def scan_body(k, pos):
        v = allids[pl.ds(k * LANES, LANES)]
        tc = v >> 7
        m = (tc & (_NW - 1)) == wid
        plsc.store_compressed(clist_id.at[pl.ds(pos, LANES)], v, mask=m)
        plsc.store_compressed(clist_pos.at[pl.ds(pos, LANES)],
                              iota + k * LANES, mask=m)
        return pos + plsc.all_reduce_population_count(m)[0]

    nkeep = lax.fori_loop(0, _NVEC, scan_body, jnp.int32(0), unroll=False)

    zeros = jnp.zeros((LANES,), jnp.int32)
    for b in range(256 // LANES):
        begin_v[pl.ds(b * LANES, LANES)] = zeros
    nchunk = (nkeep + LANES - 1) // LANES

    def hist_body(k, _):
        v = clist_id[pl.ds(k * LANES, LANES)]
        slot = v >> 12
        m = (iota + k * LANES) < nkeep
        plsc.addupdate_scatter(begin_v, [slot], ones, mask=m)
        return _

    lax.fori_loop(0, nchunk, hist_body, jnp.int32(0), unroll=False)

    def prefix_body(b, carry):
        v = begin_v[pl.ds(b * LANES, LANES)]
        s = plsc.cumsum(v) + carry
        end_v[pl.ds(b * LANES, LANES)] = s
        begin_v[pl.ds(b * LANES, LANES)] = s - v
        return s[LANES - 1]

    lax.fori_loop(0, 256 // LANES, prefix_body, jnp.int32(0), unroll=False)

    lane0 = iota == 0

    def place_body(k, acc):
        v = clist_id[pl.ds(k * LANES, LANES)]
        p = clist_pos[pl.ds(k * LANES, LANES)]
        for lane in range(LANES):
            @pl.when(k * LANES + lane < nkeep)
            def _place():
                idv = v[lane]
                pos = p[lane]
                slot = idv >> 12
                dst = _extract_scalar(begin_v, slot)
                plsc.store_scatter(sort_id, [jnp.full((LANES,), dst, jnp.int32)],
                                   jnp.full((LANES,), idv, jnp.int32), mask=lane0)
                plsc.store_scatter(sort_pos, [jnp.full((LANES,), dst, jnp.int32)],
                                   jnp.full((LANES,), pos, jnp.int32), mask=lane0)
                plsc.addupdate_scatter(begin_v, [jnp.full((LANES,), slot, jnp.int32)],
                                       ones, mask=lane0)
        return acc

    lax.fori_loop(0, nchunk, place_body, jnp.int32(0), unroll=False)

    def enqueue(t, q):
        tcg = wid + t * _NW
        @pl.when(tcg < _NTC)
        def _():
            off = pl.multiple_of(tcg * 128, 128)
            pltpu.async_copy(tabT.at[:, pl.ds(off, 128)], stage.at[q], sems[q])

    dummy_tc = tabT.at[:, pl.ds(0, 128)]
    dummy_row = exch.at[pl.ds(0, EMBED_DIM)]

    for q in range(_NRING):
        enqueue(q, q)

    def extract_tc(t, q, wcnt):
        bp = _extract_scalar(end_v, jnp.maximum(t - 1, 0))
        b0 = lax.select(t > 0, bp, jnp.int32(0))
        b1 = _extract_scalar(end_v, t)

        def elem_body(e, wcnt):
            idv = _extract_scalar(sort_id, e)
            pos = _extract_scalar(sort_pos, e)
            col = idv & 127
            cvec = jnp.full((LANES,), col, jnp.int32)
            ws = (wcnt & (_WRING - 1)) * EMBED_DIM

            @pl.when(wcnt >= _WRING)
            def _():
                pltpu.make_async_copy(dummy_row,
                                      tmp.at[pl.ds(ws, EMBED_DIM)],
                                      sem_w).wait()

            for b in range(EMBED_DIM // LANES):
                dvec = iota + b * LANES
                vv = plsc.load_gather(stage.at[q], [dvec, cvec])
                tmp[pl.ds(ws + b * LANES, LANES)] = vv
            pltpu.async_copy(tmp.at[pl.ds(ws, EMBED_DIM)],
                             exch.at[pl.ds(pos * EMBED_DIM, EMBED_DIM)],
                             sem_w)
            return wcnt + 1

        return lax.fori_loop(b0, b1, elem_body, wcnt, unroll=False)

    def group_body(g, wcnt):
        for q in range(_NRING):
            t = g * _NRING + q

            @pl.when(t * _NW + wid < _NTC)
            def _():
                pltpu.make_async_copy(dummy_tc, stage.at[q], sems[q]).wait()

            wcnt = lax.cond(t < _TPW,
                            lambda w: extract_tc(t, q, w),
                            lambda w: w, wcnt)
            enqueue(t + _NRING, q)
        return wcnt

    wcnt = lax.fori_loop(0, _NGRP, group_body, jnp.int32(0), unroll=False)

    def drain_body(d, acc):
        @pl.when(d < wcnt)
        def _drain():
            pltpu.make_async_copy(dummy_row, tmp.at[pl.ds(0, EMBED_DIM)],
                                  sem_w).wait()
        return acc

    lax.fori_loop(0, _WRING, drain_body, jnp.int32(0), unroll=False)


@functools.partial(
    pl.kernel,
    mesh=plsc.VectorSubcoreMesh(core_axis_name="c", subcore_axis_name="s"),
    out_type=(
        jax.ShapeDtypeStruct((BATCH * EMBED_DIM,), jnp.float32),
        jax.ShapeDtypeStruct((BATCH * EMBED_DIM,), jnp.float32),
    ),
    scratch_types=[
        pltpu.VMEM((BATCH,), jnp.int32),
        pltpu.VMEM((BATCH + LANES,), jnp.int32),
        pltpu.VMEM((BATCH + LANES,), jnp.int32),
        pltpu.VMEM((BATCH + LANES,), jnp.int32),
        pltpu.VMEM((BATCH + LANES,), jnp.int32),
        pltpu.VMEM((256 + LANES,), jnp.int32),
        pltpu.VMEM((256 + LANES,), jnp.int32),
        pltpu.VMEM((_NRING, EMBED_DIM, 128), jnp.float32),
        pltpu.VMEM((_WRING * EMBED_DIM,), jnp.float32),
        [pltpu.SemaphoreType.DMA] * _NRING,
        pltpu.SemaphoreType.DMA,
    ],
    compiler_params=pltpu.CompilerParams(needs_layout_passes=False),
)
def _gather_kernel(uids, iids, utabT, itabT, exch_u, exch_i,
                   allids, clist_id, clist_pos, sort_id, sort_pos,
                   begin_v, end_v, stage, tmp, sems, sem_w):
    wid = lax.axis_index("s") * _NC + lax.axis_index("c")
    pltpu.sync_copy(uids, allids)
    _gather_one(wid, allids, utabT, exch_u, clist_id, clist_pos,
                sort_id, sort_pos, begin_v, end_v, stage, tmp, sems, sem_w)
    pltpu.sync_copy(iids, allids)
    _gather_one(wid, allids, itabT, exch_i, clist_id, clist_pos,
                sort_id, sort_pos, begin_v, end_v, stage, tmp, sems, sem_w)


@functools.partial(
    pl.kernel,
    mesh=plsc.VectorSubcoreMesh(core_axis_name="c", subcore_axis_name="s"),
    out_type=(
        jax.ShapeDtypeStruct((EMBED_DIM, BATCH), jnp.float32),
        jax.ShapeDtypeStruct((EMBED_DIM, BATCH), jnp.float32),
    ),
    scratch_types=[
        pltpu.VMEM((_BPW * EMBED_DIM,), jnp.float32),
        pltpu.VMEM((EMBED_DIM, _BPW), jnp.float32),
        pltpu.SemaphoreType.DMA,
    ],
    compiler_params=pltpu.CompilerParams(needs_layout_passes=False),
)
def _transpose_kernel(exch_u, exch_i, uoutT, ioutT, buf, colsT, sem_w):
    wid = lax.axis_index("s") * _NC + lax.axis_index("c")
    base = wid * _BPW
    iota = lax.iota(jnp.int32, LANES)
    for src, dst in ((exch_u, uoutT), (exch_i, ioutT)):
        pltpu.sync_copy(src.at[pl.ds(base * EMBED_DIM, _BPW * EMBED_DIM)], buf)
        for d in range(EMBED_DIM):
            def tr_body(b, _, d=d):
                idxv = (iota + b * LANES) * EMBED_DIM + d
                v = plsc.load_gather(buf, [idxv])
                colsT[d, pl.ds(b * LANES, LANES)] = v
                return _
            lax.fori_loop(0, _BPW // LANES, tr_body, jnp.int32(0),
                          unroll=False)
        pltpu.sync_copy(colsT, dst.at[:, pl.ds(base, _BPW)])


def kernel(user_ids, item_ids, user_table, item_table):
    exch_u, exch_i = _gather_kernel(user_ids, item_ids,
                                    user_table.T, item_table.T)
    uT, iT = _transpose_kernel(exch_u, exch_i)
    return (uT.T, iT.T)
